# Initial kernel scaffold; baseline (speedup 1.0000x reference)
import jax
import jax.numpy as jnp
from jax.experimental import pallas as pl


def kernel(xyz, feature, index, W1, b1, g1, be1, W2, b2, g2, be2, W3, b3, Wl, bl):
    raise NotImplementedError("write your pallas kernel here")


# speedup vs baseline: 6.6924x; 6.6924x over previous
import functools

import jax
import jax.numpy as jnp
from jax import lax
from jax.experimental import pallas as pl
from jax.experimental.pallas import tpu as pltpu
from jax.experimental.pallas import tpu_sc as plsc

B, N, F = 4, 4096, 64
S = 512
K = 32
W_OUT = 16
C_OUT = 128
HALF = 0.1
BS = B * S

NW = 32
CPW = BS // NW
SPB = S // CPW
NCH = N // 16


def _sc_body(xyzT_hbm, index_hbm, feat_hbm, gxn_hbm, gfeat_hbm,
             xv, yv, zv, idx_v, cx_v, cy_v, cz_v,
             gi_v, giabs_v, gxn_v, gyn_v, gzn_v, rows_v, sem):
    cid = lax.axis_index("c")
    sid = lax.axis_index("s")
    wid = sid * 2 + cid
    b = wid // SPB
    s0 = (wid % SPB) * CPW

    pltpu.sync_copy(xyzT_hbm.at[pl.ds(b * 3 * N, N)], xv)
    pltpu.sync_copy(xyzT_hbm.at[pl.ds((b * 3 + 1) * N, N)], yv)
    pltpu.sync_copy(xyzT_hbm.at[pl.ds((b * 3 + 2) * N, N)], zv)
    pltpu.sync_copy(index_hbm.at[pl.ds(wid * CPW, CPW)], idx_v)

    for c in range(CPW // 16):
        ii = idx_v[pl.ds(c * 16, 16)]
        cx_v[pl.ds(c * 16, 16)] = plsc.load_gather(xv, [ii])
        cy_v[pl.ds(c * 16, 16)] = plsc.load_gather(yv, [ii])
        cz_v[pl.ds(c * 16, 16)] = plsc.load_gather(zv, [ii])

    iota = lax.iota(jnp.int32, 16)

    def center_body(j, carry):
        j16 = jnp.full((16,), j, jnp.int32)
        cxs = plsc.load_gather(cx_v, [j16])
        cys = plsc.load_gather(cy_v, [j16])
        czs = plsc.load_gather(cz_v, [j16])
        gbase = j * K

        def scan_body(i, cur):
            base = i * 16
            px = xv[pl.ds(base, 16)]
            py = yv[pl.ds(base, 16)]
            pz = zv[pl.ds(base, 16)]
            d = jnp.maximum(jnp.maximum(jnp.abs(px - cxs), jnp.abs(py - cys)),
                            jnp.abs(pz - czs))
            m = d <= HALF
            pc = jnp.sum(m.astype(jnp.int32))

            @pl.when(jnp.logical_and(cur < K, pc > 0))
            def _():
                plsc.store_compressed(gi_v.at[pl.ds(gbase + cur, 16)],
                                      base + iota, mask=m)

            return cur + pc

        cur = lax.fori_loop(0, NCH, scan_body, jnp.int32(0))
        cnt = jnp.minimum(cur, K)
        cnts = jnp.full((16,), cnt, jnp.int32)
        first = plsc.load_gather(gi_v, [jnp.full((16,), gbase, jnp.int32)])
        for c in range(K // 16):
            pos = iota + c * 16
            g = gi_v[pl.ds(gbase + c * 16, 16)]
            gi_v[pl.ds(gbase + c * 16, 16)] = jnp.where(pos < cnts, g, first)
        return carry

    lax.fori_loop(0, CPW, center_body, jnp.int32(0))

    boff = jnp.full((16,), b * N, jnp.int32)

    def kmaj_body(k, carry):
        for c in range(CPW // 16):
            jvec = iota + c * 16
            g = plsc.load_gather(gi_v, [jvec * K + k])
            gx = plsc.load_gather(xv, [g]) - cx_v[pl.ds(c * 16, 16)]
            gy = plsc.load_gather(yv, [g]) - cy_v[pl.ds(c * 16, 16)]
            gz = plsc.load_gather(zv, [g]) - cz_v[pl.ds(c * 16, 16)]
            gxn_v[k, pl.ds(c * 16, 16)] = gx
            gyn_v[k, pl.ds(c * 16, 16)] = gy
            gzn_v[k, pl.ds(c * 16, 16)] = gz
            giabs_v[pl.ds(k * CPW + c * 16, 16)] = g + boff
        return carry

    lax.fori_loop(0, K, kmaj_body, jnp.int32(0))

    pltpu.sync_copy(gxn_v, gxn_hbm.at[0, :, pl.ds(wid * CPW, CPW)])
    pltpu.sync_copy(gyn_v, gxn_hbm.at[1, :, pl.ds(wid * CPW, CPW)])
    pltpu.sync_copy(gzn_v, gxn_hbm.at[2, :, pl.ds(wid * CPW, CPW)])

    for r in range(K * CPW // 128):
        pltpu.async_copy(feat_hbm.at[giabs_v.at[pl.ds(r * 128, 128)]],
                         rows_v, sem).wait()
        k0 = 2 * r
        pltpu.sync_copy(rows_v.at[pl.ds(0, CPW)],
                        gfeat_hbm.at[k0, pl.ds(wid * CPW, CPW)])
        pltpu.sync_copy(rows_v.at[pl.ds(CPW, CPW)],
                        gfeat_hbm.at[k0 + 1, pl.ds(wid * CPW, CPW)])


_sc_query_gather = functools.partial(
    pl.kernel,
    out_type=[
        jax.ShapeDtypeStruct((3, K, BS), jnp.float32),
        jax.ShapeDtypeStruct((K, BS, F), jnp.float32),
    ],
    mesh=plsc.VectorSubcoreMesh(core_axis_name="c", subcore_axis_name="s"),
    compiler_params=pltpu.CompilerParams(use_tc_tiling_on_sc=False,
                                         needs_layout_passes=False),
    scratch_types=[
        pltpu.VMEM((N,), jnp.float32),
        pltpu.VMEM((N,), jnp.float32),
        pltpu.VMEM((N,), jnp.float32),
        pltpu.VMEM((CPW,), jnp.int32),
        pltpu.VMEM((CPW,), jnp.float32),
        pltpu.VMEM((CPW,), jnp.float32),
        pltpu.VMEM((CPW,), jnp.float32),
        pltpu.VMEM((CPW * K + 16,), jnp.int32),
        pltpu.VMEM((CPW * K,), jnp.int32),
        pltpu.VMEM((K, CPW), jnp.float32),
        pltpu.VMEM((K, CPW), jnp.float32),
        pltpu.VMEM((K, CPW), jnp.float32),
        pltpu.VMEM((128, F), jnp.float32),
        pltpu.SemaphoreType.DMA,
    ],
)(_sc_body)


BLK = 256
GRID = BS // BLK


def _tc_body(gxn_ref, gf_ref, w1_ref, b1_ref, w2_ref, b2_ref, w3_ref, b3_ref,
             wlp_ref, bl_ref, out_ref):
    gx = gxn_ref[0]
    gy = gxn_ref[1]
    gz = gxn_ref[2]
    h1 = [jax.nn.relu(gx * w1_ref[0, j] + gy * w1_ref[1, j] + gz * w1_ref[2, j]
                      + b1_ref[0, j]) for j in range(6)]
    h2 = []
    for j in range(6):
        a = b2_ref[0, j]
        t = h1[0] * w2_ref[0, j]
        for i in range(1, 6):
            t = t + h1[i] * w2_ref[i, j]
        h2.append(jax.nn.relu(t + a))
    wch = []
    for j in range(W_OUT):
        t = h2[0] * w3_ref[0, j]
        for i in range(1, 6):
            t = t + h2[i] * w3_ref[i, j]
        wch.append(t + b3_ref[0, j])

    gf = gf_ref[...].reshape(K * BLK, F)
    gft = gf.T

    acc = jnp.zeros((BLK, C_OUT), jnp.float32)
    for w in range(W_OUT):
        ww = wch[w]
        a = gft[:, 0:BLK] * ww[0:1, :]
        for k in range(1, K):
            a = a + gft[:, k * BLK:(k + 1) * BLK] * ww[k:k + 1, :]
        acc = acc + lax.dot_general(a, wlp_ref[w],
                                    (((0,), (0,)), ((), ())),
                                    precision=lax.Precision.HIGHEST,
                                    preferred_element_type=jnp.float32)
    out_ref[...] = acc + bl_ref[...]


def _tc_call(gxn, gfeat, W1f, b1f, W2f, b2f, W3, b3, WlP, bl):
    smem = functools.partial(pl.BlockSpec, memory_space=pltpu.SMEM)
    return pl.pallas_call(
        _tc_body,
        grid=(GRID,),
        in_specs=[
            pl.BlockSpec((3, K, BLK), lambda i: (0, 0, i)),
            pl.BlockSpec((K, BLK, F), lambda i: (0, i, 0)),
            smem((3, 6), lambda i: (0, 0)),
            smem((1, 6), lambda i: (0, 0)),
            smem((6, 6), lambda i: (0, 0)),
            smem((1, 6), lambda i: (0, 0)),
            smem((6, W_OUT), lambda i: (0, 0)),
            smem((1, W_OUT), lambda i: (0, 0)),
            pl.BlockSpec((W_OUT, F, C_OUT), lambda i: (0, 0, 0)),
            pl.BlockSpec((1, C_OUT), lambda i: (0, 0)),
        ],
        out_specs=pl.BlockSpec((BLK, C_OUT), lambda i: (i, 0)),
        out_shape=jax.ShapeDtypeStruct((BS, C_OUT), jnp.float32),
    )(gxn, gfeat, W1f, b1f, W2f, b2f, W3, b3, WlP, bl)


def kernel(xyz, feature, index, W1, b1, g1, be1, W2, b2, g2, be2, W3, b3,
           Wl, bl):
    xyzT = jnp.transpose(xyz, (0, 2, 1)).reshape(B * 3 * N)
    featf = feature.reshape(B * N, F)
    gxn, gfeat = _sc_query_gather(xyzT, index.reshape(BS), featf)

    inv = 1.0 / jnp.sqrt(1.0 + 1e-5)
    s1 = g1 * inv
    W1f = W1 * s1[None, :]
    b1f = (b1 * s1 + be1).reshape(1, 6)
    s2 = g2 * inv
    W2f = W2 * s2[None, :]
    b2f = (b2 * s2 + be2).reshape(1, 6)
    WlP = Wl.reshape(F, W_OUT, C_OUT).transpose(1, 0, 2)

    out = _tc_call(gxn, gfeat, W1f, b1f, W2f, b2f, W3, b3.reshape(1, W_OUT),
                   WlP, bl.reshape(1, C_OUT))
    return out.reshape(B, S, C_OUT)


# speedup vs baseline: 18.2757x; 2.7308x over previous
import functools

import jax
import jax.numpy as jnp
from jax import lax
from jax.experimental import pallas as pl
from jax.experimental.pallas import tpu as pltpu
from jax.experimental.pallas import tpu_sc as plsc

B, N, F = 4, 4096, 64
S = 512
K = 32
W_OUT = 16
C_OUT = 128
HALF = 0.1
BS = B * S

NW = 32
CPW = BS // NW
SPB = S // CPW
NCH = N // 16


def _sc_body(xyz_hbm, index_hbm, feat_hbm, gxn_hbm, gfeat_hbm,
             xraw, xv, yv, zv, idx_v, cx_v, cy_v, cz_v,
             gi_v, giabs_v, gxn_v, gyn_v, gzn_v, rows_v,
             sem0, sem1, sem2, sem3):
    cid = lax.axis_index("c")
    sid = lax.axis_index("s")
    wid = sid * 2 + cid
    b = wid // SPB

    pltpu.sync_copy(xyz_hbm.at[pl.ds(b * 3 * N, 3 * N)], xraw)
    pltpu.sync_copy(index_hbm.at[pl.ds(wid * CPW, CPW)], idx_v)

    iota = lax.iota(jnp.int32, 16)

    def deint_body(i, carry):
        base = i * 16
        p3 = (base + iota) * 3
        xv[pl.ds(base, 16)] = plsc.load_gather(xraw, [p3])
        yv[pl.ds(base, 16)] = plsc.load_gather(xraw, [p3 + 1])
        zv[pl.ds(base, 16)] = plsc.load_gather(xraw, [p3 + 2])
        return carry

    lax.fori_loop(0, NCH, deint_body, jnp.int32(0))

    for c in range(CPW // 16):
        ii = idx_v[pl.ds(c * 16, 16)]
        cx_v[pl.ds(c * 16, 16)] = plsc.load_gather(xv, [ii])
        cy_v[pl.ds(c * 16, 16)] = plsc.load_gather(yv, [ii])
        cz_v[pl.ds(c * 16, 16)] = plsc.load_gather(zv, [ii])

    G = 8

    def group_body(gidx, carry):
        j0 = gidx * G
        csp = []
        for g in range(G):
            jg = jnp.full((16,), j0 + g, jnp.int32)
            csp.append((plsc.load_gather(cx_v, [jg]),
                        plsc.load_gather(cy_v, [jg]),
                        plsc.load_gather(cz_v, [jg]),
                        jg * K, jg * K - 1))

        def scan_body(i, curs):
            base = i * 16
            px = xv[pl.ds(base, 16)]
            py = yv[pl.ds(base, 16)]
            pz = zv[pl.ds(base, 16)]
            pidx = base + iota
            new = []
            for g in range(G):
                cxs, cys, czs, gb16, gbm1 = csp[g]
                d = jnp.maximum(jnp.maximum(jnp.abs(px - cxs),
                                            jnp.abs(py - cys)),
                                jnp.abs(pz - czs))
                m = d <= HALF
                t = curs[g] + plsc.cumsum(m.astype(jnp.int32))
                wm = jnp.logical_and(m, t <= K)
                plsc.store_scatter(gi_v, [gbm1 + t], pidx, mask=wm)
                new.append(curs[g] + plsc.all_reduce_population_count(m))
            return tuple(new)

        curs = lax.fori_loop(0, NCH, scan_body,
                             tuple(jnp.zeros((16,), jnp.int32)
                                   for _ in range(G)))
        for g in range(G):
            cnts = jnp.minimum(curs[g], K)
            first = plsc.load_gather(gi_v, [csp[g][3]])
            gbase = gidx * (G * K) + g * K
            for c in range(K // 16):
                pos = iota + c * 16
                old = gi_v[pl.ds(gbase + c * 16, 16)]
                gi_v[pl.ds(gbase + c * 16, 16)] = jnp.where(pos < cnts,
                                                            old, first)
        return carry

    lax.fori_loop(0, CPW // G, group_body, jnp.int32(0))

    boff = jnp.full((16,), b * N, jnp.int32)

    def kmaj_body(k, carry):
        for c in range(CPW // 16):
            jvec = iota + c * 16
            g = plsc.load_gather(gi_v, [jvec * K + k])
            gx = plsc.load_gather(xv, [g]) - cx_v[pl.ds(c * 16, 16)]
            gy = plsc.load_gather(yv, [g]) - cy_v[pl.ds(c * 16, 16)]
            gz = plsc.load_gather(zv, [g]) - cz_v[pl.ds(c * 16, 16)]
            gxn_v[k, pl.ds(c * 16, 16)] = gx
            gyn_v[k, pl.ds(c * 16, 16)] = gy
            gzn_v[k, pl.ds(c * 16, 16)] = gz
            giabs_v[pl.ds(k * CPW + c * 16, 16)] = g + boff
        return carry

    lax.fori_loop(0, K, kmaj_body, jnp.int32(0))

    pltpu.sync_copy(gxn_v, gxn_hbm.at[0, :, pl.ds(wid * CPW, CPW)])
    pltpu.sync_copy(gyn_v, gxn_hbm.at[1, :, pl.ds(wid * CPW, CPW)])
    pltpu.sync_copy(gzn_v, gxn_hbm.at[2, :, pl.ds(wid * CPW, CPW)])

    NR = K * CPW // 128
    NB = 4
    sems = (sem0, sem1, sem2, sem3)

    def fire(r):
        return pltpu.async_copy(feat_hbm.at[giabs_v.at[pl.ds(r * 128, 128)]],
                                rows_v.at[r % NB], sems[r % NB])

    cps = [fire(r) for r in range(NB)]
    for r in range(NR):
        cps[r % NB].wait()
        k0 = 2 * r
        pltpu.sync_copy(rows_v.at[r % NB, pl.ds(0, CPW)],
                        gfeat_hbm.at[k0, pl.ds(wid * CPW, CPW)])
        pltpu.sync_copy(rows_v.at[r % NB, pl.ds(CPW, CPW)],
                        gfeat_hbm.at[k0 + 1, pl.ds(wid * CPW, CPW)])
        if r + NB < NR:
            cps[r % NB] = fire(r + NB)


_sc_query_gather = functools.partial(
    pl.kernel,
    out_type=[
        jax.ShapeDtypeStruct((3, K, BS), jnp.float32),
        jax.ShapeDtypeStruct((K, BS, F), jnp.float32),
    ],
    mesh=plsc.VectorSubcoreMesh(core_axis_name="c", subcore_axis_name="s"),
    compiler_params=pltpu.CompilerParams(use_tc_tiling_on_sc=False,
                                         needs_layout_passes=False),
    scratch_types=[
        pltpu.VMEM((3 * N,), jnp.float32),
        pltpu.VMEM((N,), jnp.float32),
        pltpu.VMEM((N,), jnp.float32),
        pltpu.VMEM((N,), jnp.float32),
        pltpu.VMEM((CPW,), jnp.int32),
        pltpu.VMEM((CPW,), jnp.float32),
        pltpu.VMEM((CPW,), jnp.float32),
        pltpu.VMEM((CPW,), jnp.float32),
        pltpu.VMEM((CPW * K + 16,), jnp.int32),
        pltpu.VMEM((CPW * K,), jnp.int32),
        pltpu.VMEM((K, CPW), jnp.float32),
        pltpu.VMEM((K, CPW), jnp.float32),
        pltpu.VMEM((K, CPW), jnp.float32),
        pltpu.VMEM((4, 128, F), jnp.float32),
        pltpu.SemaphoreType.DMA,
        pltpu.SemaphoreType.DMA,
        pltpu.SemaphoreType.DMA,
        pltpu.SemaphoreType.DMA,
    ],
)(_sc_body)


BLK = 256
GRID = BS // BLK


def _tc_body(gxn_ref, gf_ref, w1_ref, b1_ref, w2_ref, b2_ref, w3_ref, b3_ref,
             wlp_ref, bl_ref, out_ref):
    gx = gxn_ref[0]
    gy = gxn_ref[1]
    gz = gxn_ref[2]
    h1 = [jax.nn.relu(gx * w1_ref[0, j] + gy * w1_ref[1, j] + gz * w1_ref[2, j]
                      + b1_ref[0, j]) for j in range(6)]
    h2 = []
    for j in range(6):
        a = b2_ref[0, j]
        t = h1[0] * w2_ref[0, j]
        for i in range(1, 6):
            t = t + h1[i] * w2_ref[i, j]
        h2.append(jax.nn.relu(t + a))
    wch = []
    for j in range(W_OUT):
        t = h2[0] * w3_ref[0, j]
        for i in range(1, 6):
            t = t + h2[i] * w3_ref[i, j]
        wch.append(t + b3_ref[0, j])

    gf = gf_ref[...].reshape(K * BLK, F)
    gft = gf.T

    acc = jnp.zeros((BLK, C_OUT), jnp.float32)
    for w in range(W_OUT):
        ww = wch[w]
        a = gft[:, 0:BLK] * ww[0:1, :]
        for k in range(1, K):
            a = a + gft[:, k * BLK:(k + 1) * BLK] * ww[k:k + 1, :]
        acc = acc + lax.dot_general(a, wlp_ref[w],
                                    (((0,), (0,)), ((), ())),
                                    precision=lax.Precision.HIGHEST,
                                    preferred_element_type=jnp.float32)
    out_ref[...] = acc + bl_ref[...]


def _tc_call(gxn, gfeat, W1f, b1f, W2f, b2f, W3, b3, WlP, bl):
    smem = functools.partial(pl.BlockSpec, memory_space=pltpu.SMEM)
    return pl.pallas_call(
        _tc_body,
        grid=(GRID,),
        in_specs=[
            pl.BlockSpec((3, K, BLK), lambda i: (0, 0, i)),
            pl.BlockSpec((K, BLK, F), lambda i: (0, i, 0)),
            smem((3, 6), lambda i: (0, 0)),
            smem((1, 6), lambda i: (0, 0)),
            smem((6, 6), lambda i: (0, 0)),
            smem((1, 6), lambda i: (0, 0)),
            smem((6, W_OUT), lambda i: (0, 0)),
            smem((1, W_OUT), lambda i: (0, 0)),
            pl.BlockSpec((W_OUT, F, C_OUT), lambda i: (0, 0, 0)),
            pl.BlockSpec((1, C_OUT), lambda i: (0, 0)),
        ],
        out_specs=pl.BlockSpec((BLK, C_OUT), lambda i: (i, 0)),
        out_shape=jax.ShapeDtypeStruct((BS, C_OUT), jnp.float32),
    )(gxn, gfeat, W1f, b1f, W2f, b2f, W3, b3, WlP, bl)


def kernel(xyz, feature, index, W1, b1, g1, be1, W2, b2, g2, be2, W3, b3,
           Wl, bl):
    featf = feature.reshape(B * N, F)
    gxn, gfeat = _sc_query_gather(xyz.reshape(B * N * 3), index.reshape(BS),
                                  featf)

    inv = 1.0 / jnp.sqrt(1.0 + 1e-5)
    s1 = g1 * inv
    W1f = W1 * s1[None, :]
    b1f = (b1 * s1 + be1).reshape(1, 6)
    s2 = g2 * inv
    W2f = W2 * s2[None, :]
    b2f = (b2 * s2 + be2).reshape(1, 6)
    WlP = Wl.reshape(F, W_OUT, C_OUT).transpose(1, 0, 2)

    out = _tc_call(gxn, gfeat, W1f, b1f, W2f, b2f, W3, b3.reshape(1, W_OUT),
                   WlP, bl.reshape(1, C_OUT))
    return out.reshape(B, S, C_OUT)
